# Initial kernel scaffold; baseline (speedup 1.0000x reference)
import jax
import jax.numpy as jnp
from jax.experimental import pallas as pl


def kernel(x, edge_index, edge_attr, x_pos, mlp_w1, mlp_b1, mlp_w2, mlp_b2, lin1_w, lin2_w, lin2_b, lin_w, lin_b, emlp_w, emlp_b):
    raise NotImplementedError("write your pallas kernel here")


# speedup vs baseline: 1.9595x; 1.9595x over previous
import functools
import math

import jax
import jax.numpy as jnp
from jax import lax
from jax.experimental import pallas as pl
from jax.experimental.pallas import tpu as pltpu
from jax.experimental.pallas import tpu_sc as plsc

N = 10000
NP = 10240
E = 320000
D = 128
F = 256
NC = 2
NS = 16
EB = 80

_LOG2 = math.log(2.0)

_SC_PARAMS = pltpu.CompilerParams(needs_layout_passes=False)


def _ssp(v):
    return jnp.maximum(v, 0.0) + jnp.log1p(jnp.exp(-jnp.abs(v))) - _LOG2


def _sqdist_body(pos_hbm, row_hbm, col_hbm, out_hbm, pos_v, row_v, col_v, s_v,
                 sem):
    del sem
    wid = lax.axis_index("s") * NC + lax.axis_index("c")
    pltpu.sync_copy(pos_hbm, pos_v)
    epw = E // (NC * NS)
    base = wid * epw

    def blk(i, _):
        off = base + i * EB
        pltpu.sync_copy(row_hbm.at[pl.ds(off, EB)], row_v)
        pltpu.sync_copy(col_hbm.at[pl.ds(off, EB)], col_v)

        def grp(g, _):
            r = row_v[pl.ds(g * 16, 16)] * 4
            c = col_v[pl.ds(g * 16, 16)] * 4
            acc = jnp.zeros((16,), jnp.float32)
            for d in range(3):
                dv = jnp.full((16,), d, jnp.int32)
                a = plsc.load_gather(pos_v, [r + dv])
                b = plsc.load_gather(pos_v, [c + dv])
                diff = a - b
                acc = acc + diff * diff
            s_v[pl.ds(g * 16, 16)] = acc
            return 0

        lax.fori_loop(0, EB // 16, grp, 0)
        pltpu.sync_copy(s_v, out_hbm.at[pl.ds(off, EB)])
        return 0

    lax.fori_loop(0, epw // EB, blk, 0)


_sqdist = functools.partial(
    pl.kernel,
    out_type=jax.ShapeDtypeStruct((E,), jnp.float32),
    mesh=plsc.VectorSubcoreMesh(core_axis_name="c", subcore_axis_name="s"),
    scratch_types=[
        pltpu.VMEM((N * 4,), jnp.float32),
        pltpu.VMEM((EB,), jnp.int32),
        pltpu.VMEM((EB,), jnp.int32),
        pltpu.VMEM((EB,), jnp.float32),
        pltpu.SemaphoreType.DMA,
    ],
    compiler_params=_SC_PARAMS,
)(_sqdist_body)


def _scatter_body(h_hbm, w_hbm, row_hbm, col_hbm, agg_hbm,
                  row_v, col_v, cg_v, w_v, m_v, z_v, agg_sh, sem):
    c = lax.axis_index("c")
    s = lax.axis_index("s")
    rows_pt = NP // NS
    coff = c * NP

    def zrow(e, _):
        for k in range(D // 16):
            z_v[e, pl.ds(k * 16, 16)] = jnp.zeros((16,), jnp.float32)
        return 0

    lax.fori_loop(0, EB, zrow, 0)
    for b in range(rows_pt // EB):
        pltpu.sync_copy(z_v, agg_sh.at[pl.ds(s * rows_pt + b * EB, EB)])
    plsc.subcore_barrier()

    ept = E // NS

    def blk(i, _):
        off = s * ept + i * EB
        pltpu.sync_copy(row_hbm.at[pl.ds(off, EB)], row_v)
        pltpu.sync_copy(col_hbm.at[pl.ds(off, EB)], col_v)

        def grp(g, _):
            sl = pl.ds(g * 16, 16)
            cg_v[sl] = col_v[sl] + coff
            return 0

        lax.fori_loop(0, EB // 16, grp, 0)
        pltpu.async_copy(h_hbm.at[cg_v], m_v, sem).wait()
        pltpu.sync_copy(w_hbm.at[pl.ds(c * E + off, EB)], w_v)

        def mul(e, _):
            for k in range(D // 16):
                sl = pl.ds(k * 16, 16)
                m_v[e, sl] = m_v[e, sl] * w_v[e, sl]
            return 0

        lax.fori_loop(0, EB, mul, 0)
        pltpu.sync_copy(m_v, agg_sh.at[row_v], add=True)
        return 0

    lax.fori_loop(0, ept // EB, blk, 0)
    plsc.subcore_barrier()
    pltpu.sync_copy(agg_sh.at[pl.ds(s * rows_pt, rows_pt)],
                    agg_hbm.at[pl.ds(coff + s * rows_pt, rows_pt)])


_scatter = functools.partial(
    pl.kernel,
    out_type=jax.ShapeDtypeStruct((2 * NP, D), jnp.float32),
    mesh=plsc.VectorSubcoreMesh(core_axis_name="c", subcore_axis_name="s"),
    scratch_types=[
        pltpu.VMEM((EB,), jnp.int32),
        pltpu.VMEM((EB,), jnp.int32),
        pltpu.VMEM((EB,), jnp.int32),
        pltpu.VMEM((EB, D), jnp.float32),
        pltpu.VMEM((EB, D), jnp.float32),
        pltpu.VMEM((EB, D), jnp.float32),
        pltpu.VMEM_SHARED((NP, D), jnp.float32),
        pltpu.SemaphoreType.DMA,
    ],
    compiler_params=_SC_PARAMS,
)(_scatter_body)


def _egather_body(xo_hbm, row_hbm, col_hbm, g_hbm, row_v, col_v, a_v, b_v,
                  sem_a, sem_b):
    wid = lax.axis_index("s") * NC + lax.axis_index("c")
    epw = E // (NC * NS)
    base = wid * epw

    def blk(i, _):
        off = base + i * EB
        pltpu.sync_copy(row_hbm.at[pl.ds(off, EB)], row_v)
        pltpu.sync_copy(col_hbm.at[pl.ds(off, EB)], col_v)
        cp_a = pltpu.async_copy(xo_hbm.at[row_v], a_v, sem_a)
        cp_b = pltpu.async_copy(xo_hbm.at[col_v], b_v, sem_b)
        cp_a.wait()
        cp_b.wait()

        def addr(e, _):
            for k in range(D // 16):
                sl = pl.ds(k * 16, 16)
                a_v[e, sl] = a_v[e, sl] + b_v[e, sl]
            return 0

        lax.fori_loop(0, EB, addr, 0)
        pltpu.sync_copy(a_v, g_hbm.at[pl.ds(off, EB)])
        return 0

    lax.fori_loop(0, epw // EB, blk, 0)


_egather = functools.partial(
    pl.kernel,
    out_type=jax.ShapeDtypeStruct((E, D), jnp.float32),
    mesh=plsc.VectorSubcoreMesh(core_axis_name="c", subcore_axis_name="s"),
    scratch_types=[
        pltpu.VMEM((EB,), jnp.int32),
        pltpu.VMEM((EB,), jnp.int32),
        pltpu.VMEM((EB, D), jnp.float32),
        pltpu.VMEM((EB, D), jnp.float32),
        pltpu.SemaphoreType.DMA,
        pltpu.SemaphoreType.DMA,
    ],
    compiler_params=_SC_PARAMS,
)(_egather_body)


BN = 512
BE = 1280


def _h_body(x_ref, w_ref, o_ref):
    o_ref[0] = jnp.dot(x_ref[...], w_ref[...],
                       preferred_element_type=jnp.float32)


_h_call = pl.pallas_call(
    _h_body,
    grid=(NP // BN, 2),
    in_specs=[
        pl.BlockSpec((BN, D), lambda i, j: (i, 0)),
        pl.BlockSpec((D, D), lambda i, j: (0, j)),
    ],
    out_specs=pl.BlockSpec((1, BN, D), lambda i, j: (j, i, 0)),
    out_shape=jax.ShapeDtypeStruct((2, NP, D), jnp.float32),
)


def _w_body(ea_ref, s_ref, w1_ref, b1_ref, w2_ref, b2_ref, o_ref):
    t = _ssp(jnp.dot(ea_ref[...], w1_ref[...],
                     preferred_element_type=jnp.float32) + b1_ref[...])
    wf = jnp.dot(t, w2_ref[...], preferred_element_type=jnp.float32) + b2_ref[...]
    dist = jnp.sqrt(s_ref[...] + 1e-12)
    cfac = 0.5 * (jnp.cos(dist * (math.pi / 10.0)) + 1.0)
    cfac = cfac * (dist < 10.0).astype(jnp.float32)
    wf = wf * cfac
    o_ref[0] = wf[:, :D]
    o_ref[1] = wf[:, D:]


_w_call = pl.pallas_call(
    _w_body,
    grid=(E // BE,),
    in_specs=[
        pl.BlockSpec((BE, D), lambda i: (i, 0)),
        pl.BlockSpec((BE, 1), lambda i: (i, 0)),
        pl.BlockSpec((D, F), lambda i: (0, 0)),
        pl.BlockSpec((1, F), lambda i: (0, 0)),
        pl.BlockSpec((F, F), lambda i: (0, 0)),
        pl.BlockSpec((1, F), lambda i: (0, 0)),
    ],
    out_specs=pl.BlockSpec((2, BE, D), lambda i: (0, i, 0)),
    out_shape=jax.ShapeDtypeStruct((2, E, D), jnp.float32),
)


def _post_body(a0_ref, a1_ref, x_ref, l2a_ref, l2b_ref, b2_ref, lw_ref, lb_ref,
               o_ref):
    t = (jnp.dot(a0_ref[...], l2a_ref[...], preferred_element_type=jnp.float32)
         + jnp.dot(a1_ref[...], l2b_ref[...], preferred_element_type=jnp.float32)
         + b2_ref[...])
    conv = jnp.dot(_ssp(t), lw_ref[...],
                   preferred_element_type=jnp.float32) + lb_ref[...]
    o_ref[...] = jnp.maximum(conv, 0.0) + x_ref[...]


_post_call = pl.pallas_call(
    _post_body,
    grid=(NP // BN,),
    in_specs=[
        pl.BlockSpec((BN, D), lambda i: (i, 0)),
        pl.BlockSpec((BN, D), lambda i: (i, 0)),
        pl.BlockSpec((BN, D), lambda i: (i, 0)),
        pl.BlockSpec((D, D), lambda i: (0, 0)),
        pl.BlockSpec((D, D), lambda i: (0, 0)),
        pl.BlockSpec((1, D), lambda i: (0, 0)),
        pl.BlockSpec((D, D), lambda i: (0, 0)),
        pl.BlockSpec((1, D), lambda i: (0, 0)),
    ],
    out_specs=pl.BlockSpec((BN, D), lambda i: (i, 0)),
    out_shape=jax.ShapeDtypeStruct((NP, D), jnp.float32),
)


def _edge_body(ea_ref, g_ref, wa_ref, wb_ref, b_ref, o_ref):
    t = (jnp.dot(ea_ref[...], wa_ref[...], preferred_element_type=jnp.float32)
         + jnp.dot(g_ref[...], wb_ref[...], preferred_element_type=jnp.float32)
         + b_ref[...])
    o_ref[...] = jnp.tanh(t) + ea_ref[...]


_edge_call = pl.pallas_call(
    _edge_body,
    grid=(E // BE,),
    in_specs=[
        pl.BlockSpec((BE, D), lambda i: (i, 0)),
        pl.BlockSpec((BE, D), lambda i: (i, 0)),
        pl.BlockSpec((D, D), lambda i: (0, 0)),
        pl.BlockSpec((D, D), lambda i: (0, 0)),
        pl.BlockSpec((1, D), lambda i: (0, 0)),
    ],
    out_specs=pl.BlockSpec((BE, D), lambda i: (i, 0)),
    out_shape=jax.ShapeDtypeStruct((E, D), jnp.float32),
)


@jax.jit
def kernel(x, edge_index, edge_attr, x_pos, mlp_w1, mlp_b1, mlp_w2, mlp_b2,
           lin1_w, lin2_w, lin2_b, lin_w, lin_b, emlp_w, emlp_b):
    row = edge_index[0]
    col = edge_index[1]
    pos4 = jnp.pad(x_pos, ((0, 0), (0, 1)))
    xp = jnp.pad(x, ((0, NP - N), (0, 0)))

    s = _sqdist(pos4.reshape(-1), row, col)
    h2 = _h_call(xp, lin1_w).reshape(2 * NP, D)
    w2 = _w_call(edge_attr, s.reshape(E, 1), mlp_w1, mlp_b1.reshape(1, F),
                 mlp_w2, mlp_b2.reshape(1, F)).reshape(2 * E, D)
    agg2 = _scatter(h2, w2, row, col)
    xo = _post_call(agg2[:NP], agg2[NP:], xp, lin2_w[:D], lin2_w[D:],
                    lin2_b.reshape(1, D), lin_w, lin_b.reshape(1, D))
    g = _egather(xo, row, col)
    eo = _edge_call(edge_attr, g, emlp_w[:D], emlp_w[D:],
                    emlp_b.reshape(1, D))
    return (xo[:N], eo)


# speedup vs baseline: 2.9471x; 1.5040x over previous
import functools
import math

import jax
import jax.numpy as jnp
from jax import lax
from jax.experimental import pallas as pl
from jax.experimental.pallas import tpu as pltpu
from jax.experimental.pallas import tpu_sc as plsc

N = 10000
NP = 10240
E = 320000
D = 128
F = 256
NC = 2
NS = 16
EB = 80

_LOG2 = math.log(2.0)

_SC_PARAMS = pltpu.CompilerParams(needs_layout_passes=False)


def _ssp(v):
    return jnp.maximum(v, 0.0) + jnp.log1p(jnp.exp(-jnp.abs(v))) - _LOG2


def _sqdist_body(pos_hbm, row_hbm, col_hbm, out_hbm, pos_v, row_v, col_v, s_v,
                 sem):
    del sem
    wid = lax.axis_index("s") * NC + lax.axis_index("c")
    pltpu.sync_copy(pos_hbm, pos_v)
    epw = E // (NC * NS)
    base = wid * epw
    pltpu.sync_copy(row_hbm.at[pl.ds(base, epw)], row_v)
    pltpu.sync_copy(col_hbm.at[pl.ds(base, epw)], col_v)

    def grp(g, _):
        r = row_v[pl.ds(g * 16, 16)] * 4
        c = col_v[pl.ds(g * 16, 16)] * 4
        acc = jnp.zeros((16,), jnp.float32)
        for d in range(3):
            dv = jnp.full((16,), d, jnp.int32)
            a = plsc.load_gather(pos_v, [r + dv])
            b = plsc.load_gather(pos_v, [c + dv])
            diff = a - b
            acc = acc + diff * diff
        s_v[pl.ds(g * 16, 16)] = acc
        return 0

    lax.fori_loop(0, epw // 16, grp, 0)
    pltpu.sync_copy(s_v, out_hbm.at[pl.ds(base, epw)])


_sqdist = functools.partial(
    pl.kernel,
    out_type=jax.ShapeDtypeStruct((E,), jnp.float32),
    mesh=plsc.VectorSubcoreMesh(core_axis_name="c", subcore_axis_name="s"),
    scratch_types=[
        pltpu.VMEM((N * 4,), jnp.float32),
        pltpu.VMEM((E // (NC * NS),), jnp.int32),
        pltpu.VMEM((E // (NC * NS),), jnp.int32),
        pltpu.VMEM((E // (NC * NS),), jnp.float32),
        pltpu.SemaphoreType.DMA,
    ],
    compiler_params=_SC_PARAMS,
)(_sqdist_body)


_SB = 25
_NSB = E // NS // EB // _SB


def _scatter_body(h_hbm, w_hbm, row_hbm, col_hbm, agg_hbm,
                  row_sb, cg_sb, w_v0, w_v1, m_v0, m_v1, agg_sh,
                  sem_g0, sem_g1, sem_w0, sem_w1):
    c = lax.axis_index("c")
    s = lax.axis_index("s")
    rows_pt = NP // NS
    coff = c * NP
    ept = E // NS

    def zrow(e, _):
        for k in range(D // 16):
            m_v0[e, pl.ds(k * 16, 16)] = jnp.zeros((16,), jnp.float32)
        return 0

    lax.fori_loop(0, EB, zrow, 0)
    for b in range(rows_pt // EB):
        pltpu.sync_copy(m_v0, agg_sh.at[pl.ds(s * rows_pt + b * EB, EB)])
    plsc.subcore_barrier()

    w_bufs = (w_v0, w_v1)
    m_bufs = (m_v0, m_v1)
    sem_gs = (sem_g0, sem_g1)
    sem_ws = (sem_w0, sem_w1)

    def issue(sbi, t, b):
        off = c * E + s * ept + (sbi * _SB + t) * EB
        pltpu.async_copy(h_hbm.at[cg_sb.at[t]], m_bufs[b], sem_gs[b])
        pltpu.async_copy(w_hbm.at[pl.ds(off, EB)], w_bufs[b], sem_ws[b])

    def sblk(sbi, _):
        pltpu.sync_copy(row_hbm.at[s, sbi], row_sb)
        pltpu.sync_copy(col_hbm.at[s, sbi], cg_sb)

        def gblk(i, _):
            for g in range(EB // 16):
                sl = pl.ds(g * 16, 16)
                cg_sb[i, sl] = cg_sb[i, sl] + coff
            return 0

        lax.fori_loop(0, _SB, gblk, 0)

        issue(sbi, 0, 0)
        for t in range(_SB):
            b = t % 2
            if t + 1 < _SB:
                issue(sbi, t + 1, 1 - b)
            m_v = m_bufs[b]
            w_v = w_bufs[b]
            pltpu.make_async_copy(h_hbm.at[cg_sb.at[t]], m_v,
                                  sem_gs[b]).wait()
            pltpu.make_async_copy(
                w_hbm.at[pl.ds(c * E + s * ept + (sbi * _SB + t) * EB, EB)],
                w_v, sem_ws[b]).wait()

            def mul(e, _):
                for q in range(D // 16):
                    sl = pl.ds(q * 16, 16)
                    m_v[e, sl] = m_v[e, sl] * w_v[e, sl]
                return 0

            lax.fori_loop(0, EB, mul, 0)
            pltpu.sync_copy(m_v, agg_sh.at[row_sb.at[t]], add=True)
        return 0

    lax.fori_loop(0, _NSB, sblk, 0)
    plsc.subcore_barrier()
    pltpu.sync_copy(agg_sh.at[pl.ds(s * rows_pt, rows_pt)],
                    agg_hbm.at[pl.ds(coff + s * rows_pt, rows_pt)])


_scatter = functools.partial(
    pl.kernel,
    out_type=jax.ShapeDtypeStruct((2 * NP, D), jnp.float32),
    mesh=plsc.VectorSubcoreMesh(core_axis_name="c", subcore_axis_name="s"),
    scratch_types=[
        pltpu.VMEM((_SB, EB), jnp.int32),
        pltpu.VMEM((_SB, EB), jnp.int32),
        pltpu.VMEM((EB, D), jnp.float32),
        pltpu.VMEM((EB, D), jnp.float32),
        pltpu.VMEM((EB, D), jnp.float32),
        pltpu.VMEM((EB, D), jnp.float32),
        pltpu.VMEM_SHARED((NP, D), jnp.float32),
        pltpu.SemaphoreType.DMA,
        pltpu.SemaphoreType.DMA,
        pltpu.SemaphoreType.DMA,
        pltpu.SemaphoreType.DMA,
    ],
    compiler_params=_SC_PARAMS,
)(_scatter_body)


_NBLK_EG = E // (NC * NS) // EB


def _egather_body(xo_hbm, row_hbm, col_hbm, g_hbm, row2d, col2d,
                  a_v0, a_v1, b_v0, b_v1,
                  sem_a0, sem_a1, sem_b0, sem_b1):
    wid = lax.axis_index("s") * NC + lax.axis_index("c")
    epw = E // (NC * NS)
    base = wid * epw
    pltpu.sync_copy(row_hbm.at[wid], row2d)
    pltpu.sync_copy(col_hbm.at[wid], col2d)

    a_bufs = (a_v0, a_v1)
    b_bufs = (b_v0, b_v1)
    sem_as = (sem_a0, sem_a1)
    sem_bs = (sem_b0, sem_b1)

    def issue(k, b):
        pltpu.async_copy(xo_hbm.at[row2d.at[k]], a_bufs[b], sem_as[b])
        pltpu.async_copy(xo_hbm.at[col2d.at[k]], b_bufs[b], sem_bs[b])

    issue(0, 0)

    def blk(j, _):
        for b in range(2):
            k = 2 * j + b
            pl.when(k + 1 < _NBLK_EG)(
                functools.partial(issue, k + 1, 1 - b))
            a_v = a_bufs[b]
            b_v = b_bufs[b]
            pltpu.make_async_copy(xo_hbm.at[row2d.at[k]], a_v,
                                  sem_as[b]).wait()
            pltpu.make_async_copy(xo_hbm.at[col2d.at[k]], b_v,
                                  sem_bs[b]).wait()

            def addr(e, _):
                for q in range(D // 16):
                    sl = pl.ds(q * 16, 16)
                    a_v[e, sl] = a_v[e, sl] + b_v[e, sl]
                return 0

            lax.fori_loop(0, EB, addr, 0)
            pltpu.sync_copy(a_v, g_hbm.at[pl.ds(base + k * EB, EB)])
        return 0

    lax.fori_loop(0, _NBLK_EG // 2, blk, 0)
    k_last = _NBLK_EG - 1
    a_v = a_bufs[k_last % 2]
    b_v = b_bufs[k_last % 2]
    pltpu.make_async_copy(xo_hbm.at[row2d.at[k_last]], a_v,
                          sem_as[k_last % 2]).wait()
    pltpu.make_async_copy(xo_hbm.at[col2d.at[k_last]], b_v,
                          sem_bs[k_last % 2]).wait()

    def addr_last(e, _):
        for q in range(D // 16):
            sl = pl.ds(q * 16, 16)
            a_v[e, sl] = a_v[e, sl] + b_v[e, sl]
        return 0

    lax.fori_loop(0, EB, addr_last, 0)
    pltpu.sync_copy(a_v, g_hbm.at[pl.ds(base + k_last * EB, EB)])


_egather = functools.partial(
    pl.kernel,
    out_type=jax.ShapeDtypeStruct((E, D), jnp.float32),
    mesh=plsc.VectorSubcoreMesh(core_axis_name="c", subcore_axis_name="s"),
    scratch_types=[
        pltpu.VMEM((_NBLK_EG, EB), jnp.int32),
        pltpu.VMEM((_NBLK_EG, EB), jnp.int32),
        pltpu.VMEM((EB, D), jnp.float32),
        pltpu.VMEM((EB, D), jnp.float32),
        pltpu.VMEM((EB, D), jnp.float32),
        pltpu.VMEM((EB, D), jnp.float32),
        pltpu.SemaphoreType.DMA,
        pltpu.SemaphoreType.DMA,
        pltpu.SemaphoreType.DMA,
        pltpu.SemaphoreType.DMA,
    ],
    compiler_params=_SC_PARAMS,
)(_egather_body)


BN = 512
BE = 1280


def _h_body(x_ref, w_ref, o_ref):
    o_ref[0] = jnp.dot(x_ref[...], w_ref[...],
                       preferred_element_type=jnp.float32)


_h_call = pl.pallas_call(
    _h_body,
    grid=(NP // BN, 2),
    in_specs=[
        pl.BlockSpec((BN, D), lambda i, j: (i, 0)),
        pl.BlockSpec((D, D), lambda i, j: (0, j)),
    ],
    out_specs=pl.BlockSpec((1, BN, D), lambda i, j: (j, i, 0)),
    out_shape=jax.ShapeDtypeStruct((2, NP, D), jnp.float32),
)


def _w_body(ea_ref, s_ref, w1_ref, b1_ref, w2_ref, b2_ref, o_ref):
    t = _ssp(jnp.dot(ea_ref[...], w1_ref[...],
                     preferred_element_type=jnp.float32) + b1_ref[...])
    wf = jnp.dot(t, w2_ref[...], preferred_element_type=jnp.float32) + b2_ref[...]
    dist = jnp.sqrt(s_ref[...] + 1e-12)
    cfac = 0.5 * (jnp.cos(dist * (math.pi / 10.0)) + 1.0)
    cfac = cfac * (dist < 10.0).astype(jnp.float32)
    wf = wf * cfac
    o_ref[0] = wf[:, :D]
    o_ref[1] = wf[:, D:]


_w_call = pl.pallas_call(
    _w_body,
    grid=(E // BE,),
    in_specs=[
        pl.BlockSpec((BE, D), lambda i: (i, 0)),
        pl.BlockSpec((BE, 1), lambda i: (i, 0)),
        pl.BlockSpec((D, F), lambda i: (0, 0)),
        pl.BlockSpec((1, F), lambda i: (0, 0)),
        pl.BlockSpec((F, F), lambda i: (0, 0)),
        pl.BlockSpec((1, F), lambda i: (0, 0)),
    ],
    out_specs=pl.BlockSpec((2, BE, D), lambda i: (0, i, 0)),
    out_shape=jax.ShapeDtypeStruct((2, E, D), jnp.float32),
)


def _post_body(a0_ref, a1_ref, x_ref, l2a_ref, l2b_ref, b2_ref, lw_ref, lb_ref,
               o_ref):
    t = (jnp.dot(a0_ref[...], l2a_ref[...], preferred_element_type=jnp.float32)
         + jnp.dot(a1_ref[...], l2b_ref[...], preferred_element_type=jnp.float32)
         + b2_ref[...])
    conv = jnp.dot(_ssp(t), lw_ref[...],
                   preferred_element_type=jnp.float32) + lb_ref[...]
    o_ref[...] = jnp.maximum(conv, 0.0) + x_ref[...]


_post_call = pl.pallas_call(
    _post_body,
    grid=(NP // BN,),
    in_specs=[
        pl.BlockSpec((BN, D), lambda i: (i, 0)),
        pl.BlockSpec((BN, D), lambda i: (i, 0)),
        pl.BlockSpec((BN, D), lambda i: (i, 0)),
        pl.BlockSpec((D, D), lambda i: (0, 0)),
        pl.BlockSpec((D, D), lambda i: (0, 0)),
        pl.BlockSpec((1, D), lambda i: (0, 0)),
        pl.BlockSpec((D, D), lambda i: (0, 0)),
        pl.BlockSpec((1, D), lambda i: (0, 0)),
    ],
    out_specs=pl.BlockSpec((BN, D), lambda i: (i, 0)),
    out_shape=jax.ShapeDtypeStruct((NP, D), jnp.float32),
)


def _edge_body(ea_ref, g_ref, wa_ref, wb_ref, b_ref, o_ref):
    t = (jnp.dot(ea_ref[...], wa_ref[...], preferred_element_type=jnp.float32)
         + jnp.dot(g_ref[...], wb_ref[...], preferred_element_type=jnp.float32)
         + b_ref[...])
    o_ref[...] = jnp.tanh(t) + ea_ref[...]


_edge_call = pl.pallas_call(
    _edge_body,
    grid=(E // BE,),
    in_specs=[
        pl.BlockSpec((BE, D), lambda i: (i, 0)),
        pl.BlockSpec((BE, D), lambda i: (i, 0)),
        pl.BlockSpec((D, D), lambda i: (0, 0)),
        pl.BlockSpec((D, D), lambda i: (0, 0)),
        pl.BlockSpec((1, D), lambda i: (0, 0)),
    ],
    out_specs=pl.BlockSpec((BE, D), lambda i: (i, 0)),
    out_shape=jax.ShapeDtypeStruct((E, D), jnp.float32),
)


@jax.jit
def kernel(x, edge_index, edge_attr, x_pos, mlp_w1, mlp_b1, mlp_w2, mlp_b2,
           lin1_w, lin2_w, lin2_b, lin_w, lin_b, emlp_w, emlp_b):
    row = edge_index[0]
    col = edge_index[1]
    row_sc = row.reshape(NS, _NSB, _SB, EB)
    col_sc = col.reshape(NS, _NSB, _SB, EB)
    row_eg = row.reshape(NC * NS, _NBLK_EG, EB)
    col_eg = col.reshape(NC * NS, _NBLK_EG, EB)
    pos4 = jnp.pad(x_pos, ((0, 0), (0, 1)))
    xp = jnp.pad(x, ((0, NP - N), (0, 0)))

    s = _sqdist(pos4.reshape(-1), row, col)
    h2 = _h_call(xp, lin1_w).reshape(2 * NP, D)
    w2 = _w_call(edge_attr, s.reshape(E, 1), mlp_w1, mlp_b1.reshape(1, F),
                 mlp_w2, mlp_b2.reshape(1, F)).reshape(2 * E, D)
    agg2 = _scatter(h2, w2, row_sc, col_sc)
    xo = _post_call(agg2[:NP], agg2[NP:], xp, lin2_w[:D], lin2_w[D:],
                    lin2_b.reshape(1, D), lin_w, lin_b.reshape(1, D))
    g = _egather(xo, row_eg, col_eg)
    eo = _edge_call(edge_attr, g, emlp_w[:D], emlp_w[D:],
                    emlp_b.reshape(1, D))
    return (xo[:N], eo)


# speedup vs baseline: 4.0359x; 1.3694x over previous
import functools
import math

import jax
import jax.numpy as jnp
from jax import lax
from jax.experimental import pallas as pl
from jax.experimental.pallas import tpu as pltpu
from jax.experimental.pallas import tpu_sc as plsc

N = 10000
NP = 10240
E = 320000
D = 128
F = 256
NC = 2
NS = 16
EB = 80

_LOG2 = math.log(2.0)

_SC_PARAMS = pltpu.CompilerParams(needs_layout_passes=False)


def _ssp(v):
    return jnp.maximum(v, 0.0) + jnp.log1p(jnp.exp(-jnp.abs(v))) - _LOG2


def _sqdist_body(pos_hbm, row_hbm, col_hbm, out_hbm, pos_v, row_v, col_v, s_v,
                 sem):
    del sem
    wid = lax.axis_index("s") * NC + lax.axis_index("c")
    pltpu.sync_copy(pos_hbm, pos_v)
    epw = E // (NC * NS)
    base = wid * epw
    pltpu.sync_copy(row_hbm.at[pl.ds(base, epw)], row_v)
    pltpu.sync_copy(col_hbm.at[pl.ds(base, epw)], col_v)

    def grp(g, _):
        r = row_v[pl.ds(g * 16, 16)] * 4
        c = col_v[pl.ds(g * 16, 16)] * 4
        acc = jnp.zeros((16,), jnp.float32)
        for d in range(3):
            dv = jnp.full((16,), d, jnp.int32)
            a = plsc.load_gather(pos_v, [r + dv])
            b = plsc.load_gather(pos_v, [c + dv])
            diff = a - b
            acc = acc + diff * diff
        s_v[pl.ds(g * 16, 16)] = acc
        return 0

    lax.fori_loop(0, epw // 16, grp, 0)
    pltpu.sync_copy(s_v, out_hbm.at[pl.ds(base, epw)])


_sqdist = functools.partial(
    pl.kernel,
    out_type=jax.ShapeDtypeStruct((E,), jnp.float32),
    mesh=plsc.VectorSubcoreMesh(core_axis_name="c", subcore_axis_name="s"),
    scratch_types=[
        pltpu.VMEM((N * 4,), jnp.float32),
        pltpu.VMEM((E // (NC * NS),), jnp.int32),
        pltpu.VMEM((E // (NC * NS),), jnp.int32),
        pltpu.VMEM((E // (NC * NS),), jnp.float32),
        pltpu.SemaphoreType.DMA,
    ],
    compiler_params=_SC_PARAMS,
)(_sqdist_body)


_SB = 25
_NSB = E // NS // EB // _SB


def _scatter_body(h_hbm, w_hbm, row_hbm, col_hbm, agg_hbm,
                  row_sb, cg_sb, w_v0, w_v1, m_v0, m_v1, agg_sh,
                  sem_g0, sem_g1, sem_w0, sem_w1):
    c = lax.axis_index("c")
    s = lax.axis_index("s")
    rows_pt = NP // NS
    coff = c * NP
    ept = E // NS

    def zrow(e, _):
        for k in range(D // 16):
            m_v0[e, pl.ds(k * 16, 16)] = jnp.zeros((16,), jnp.float32)
        return 0

    lax.fori_loop(0, EB, zrow, 0)
    for b in range(rows_pt // EB):
        pltpu.sync_copy(m_v0, agg_sh.at[pl.ds(s * rows_pt + b * EB, EB)])
    plsc.subcore_barrier()

    w_bufs = (w_v0, w_v1)
    m_bufs = (m_v0, m_v1)
    sem_gs = (sem_g0, sem_g1)
    sem_ws = (sem_w0, sem_w1)

    def issue(sbi, t, b):
        off = c * E + s * ept + (sbi * _SB + t) * EB
        pltpu.async_copy(h_hbm.at[cg_sb.at[t]], m_bufs[b], sem_gs[b])
        pltpu.async_copy(w_hbm.at[pl.ds(off, EB)], w_bufs[b], sem_ws[b])

    def sblk(sbi, _):
        pltpu.sync_copy(row_hbm.at[s, sbi], row_sb)
        pltpu.sync_copy(col_hbm.at[s, sbi], cg_sb)

        def gblk(i, _):
            for g in range(EB // 16):
                sl = pl.ds(g * 16, 16)
                cg_sb[i, sl] = cg_sb[i, sl] + coff
            return 0

        lax.fori_loop(0, _SB, gblk, 0)

        issue(sbi, 0, 0)
        for t in range(_SB):
            b = t % 2
            if t + 1 < _SB:
                issue(sbi, t + 1, 1 - b)
            m_v = m_bufs[b]
            w_v = w_bufs[b]
            pltpu.make_async_copy(h_hbm.at[cg_sb.at[t]], m_v,
                                  sem_gs[b]).wait()
            pltpu.make_async_copy(
                w_hbm.at[pl.ds(c * E + s * ept + (sbi * _SB + t) * EB, EB)],
                w_v, sem_ws[b]).wait()

            def mul(e, _):
                for q in range(D // 16):
                    sl = pl.ds(q * 16, 16)
                    m_v[e, sl] = m_v[e, sl] * w_v[e, sl]
                return 0

            lax.fori_loop(0, EB, mul, 0)
            pltpu.sync_copy(m_v, agg_sh.at[row_sb.at[t]], add=True)
        return 0

    lax.fori_loop(0, _NSB, sblk, 0)
    plsc.subcore_barrier()
    pltpu.sync_copy(agg_sh.at[pl.ds(s * rows_pt, rows_pt)],
                    agg_hbm.at[pl.ds(coff + s * rows_pt, rows_pt)])


_scatter = functools.partial(
    pl.kernel,
    out_type=jax.ShapeDtypeStruct((2 * NP, D), jnp.float32),
    mesh=plsc.VectorSubcoreMesh(core_axis_name="c", subcore_axis_name="s"),
    scratch_types=[
        pltpu.VMEM((_SB, EB), jnp.int32),
        pltpu.VMEM((_SB, EB), jnp.int32),
        pltpu.VMEM((EB, D), jnp.float32),
        pltpu.VMEM((EB, D), jnp.float32),
        pltpu.VMEM((EB, D), jnp.float32),
        pltpu.VMEM((EB, D), jnp.float32),
        pltpu.VMEM_SHARED((NP, D), jnp.float32),
        pltpu.SemaphoreType.DMA,
        pltpu.SemaphoreType.DMA,
        pltpu.SemaphoreType.DMA,
        pltpu.SemaphoreType.DMA,
    ],
    compiler_params=_SC_PARAMS,
)(_scatter_body)


_NBLK_EG = E // (NC * NS) // EB


def _egather_body(xo_hbm, row_hbm, col_hbm, g_hbm, row2d, col2d,
                  a_v0, a_v1, b_v0, b_v1,
                  sem_a0, sem_a1, sem_b0, sem_b1):
    wid = lax.axis_index("s") * NC + lax.axis_index("c")
    epw = E // (NC * NS)
    base = wid * epw
    pltpu.sync_copy(row_hbm.at[wid], row2d)
    pltpu.sync_copy(col_hbm.at[wid], col2d)

    a_bufs = (a_v0, a_v1)
    b_bufs = (b_v0, b_v1)
    sem_as = (sem_a0, sem_a1)
    sem_bs = (sem_b0, sem_b1)

    def issue(k, b):
        pltpu.async_copy(xo_hbm.at[row2d.at[k]], a_bufs[b], sem_as[b])
        pltpu.async_copy(xo_hbm.at[col2d.at[k]], b_bufs[b], sem_bs[b])

    issue(0, 0)

    def blk(j, _):
        for b in range(2):
            k = 2 * j + b
            pl.when(k + 1 < _NBLK_EG)(
                functools.partial(issue, k + 1, 1 - b))
            a_v = a_bufs[b]
            b_v = b_bufs[b]
            pltpu.make_async_copy(xo_hbm.at[row2d.at[k]], a_v,
                                  sem_as[b]).wait()
            pltpu.make_async_copy(xo_hbm.at[col2d.at[k]], b_v,
                                  sem_bs[b]).wait()

            def addr(e, _):
                for q in range(D // 16):
                    sl = pl.ds(q * 16, 16)
                    a_v[e, sl] = a_v[e, sl] + b_v[e, sl]
                return 0

            lax.fori_loop(0, EB, addr, 0)
            pltpu.sync_copy(a_v, g_hbm.at[pl.ds(base + k * EB, EB)])
        return 0

    lax.fori_loop(0, _NBLK_EG // 2, blk, 0)
    k_last = _NBLK_EG - 1
    a_v = a_bufs[k_last % 2]
    b_v = b_bufs[k_last % 2]
    pltpu.make_async_copy(xo_hbm.at[row2d.at[k_last]], a_v,
                          sem_as[k_last % 2]).wait()
    pltpu.make_async_copy(xo_hbm.at[col2d.at[k_last]], b_v,
                          sem_bs[k_last % 2]).wait()

    def addr_last(e, _):
        for q in range(D // 16):
            sl = pl.ds(q * 16, 16)
            a_v[e, sl] = a_v[e, sl] + b_v[e, sl]
        return 0

    lax.fori_loop(0, EB, addr_last, 0)
    pltpu.sync_copy(a_v, g_hbm.at[pl.ds(base + k_last * EB, EB)])


_egather = functools.partial(
    pl.kernel,
    out_type=jax.ShapeDtypeStruct((E, D), jnp.float32),
    mesh=plsc.VectorSubcoreMesh(core_axis_name="c", subcore_axis_name="s"),
    scratch_types=[
        pltpu.VMEM((_NBLK_EG, EB), jnp.int32),
        pltpu.VMEM((_NBLK_EG, EB), jnp.int32),
        pltpu.VMEM((EB, D), jnp.float32),
        pltpu.VMEM((EB, D), jnp.float32),
        pltpu.VMEM((EB, D), jnp.float32),
        pltpu.VMEM((EB, D), jnp.float32),
        pltpu.SemaphoreType.DMA,
        pltpu.SemaphoreType.DMA,
        pltpu.SemaphoreType.DMA,
        pltpu.SemaphoreType.DMA,
    ],
    compiler_params=_SC_PARAMS,
)(_egather_body)


BN = 512
BE = 1280


def _h_body(x_ref, w_ref, o_ref):
    o_ref[0] = jnp.dot(x_ref[...], w_ref[...],
                       preferred_element_type=jnp.float32)


_h_call = pl.pallas_call(
    _h_body,
    grid=(NP // BN, 2),
    in_specs=[
        pl.BlockSpec((BN, D), lambda i, j: (i, 0)),
        pl.BlockSpec((D, D), lambda i, j: (0, j)),
    ],
    out_specs=pl.BlockSpec((1, BN, D), lambda i, j: (j, i, 0)),
    out_shape=jax.ShapeDtypeStruct((2, NP, D), jnp.float32),
)


def _cut_body(s_ref, o_ref):
    dist = jnp.sqrt(s_ref[...] + 1e-12)
    cfac = 0.5 * (jnp.cos(dist * (math.pi / 10.0)) + 1.0)
    o_ref[...] = cfac * (dist < 10.0).astype(jnp.float32)


_cut_call = pl.pallas_call(
    _cut_body,
    grid=(1,),
    in_specs=[pl.BlockSpec((E // 128, 128), lambda i: (0, 0))],
    out_specs=pl.BlockSpec((E // 128, 128), lambda i: (0, 0)),
    out_shape=jax.ShapeDtypeStruct((E // 128, 128), jnp.float32),
)


def _w_body(ea_ref, c_ref, w1_ref, b1_ref, w2_ref, b2_ref, o_ref):
    t = _ssp(jnp.dot(ea_ref[...], w1_ref[...],
                     preferred_element_type=jnp.float32) + b1_ref[...])
    wf = jnp.dot(t, w2_ref[...], preferred_element_type=jnp.float32) + b2_ref[...]
    wf = wf * c_ref[...]
    o_ref[0] = wf[:, :D]
    o_ref[1] = wf[:, D:]


_w_call = pl.pallas_call(
    _w_body,
    grid=(E // BE,),
    in_specs=[
        pl.BlockSpec((BE, D), lambda i: (i, 0)),
        pl.BlockSpec((BE, 1), lambda i: (i, 0)),
        pl.BlockSpec((D, F), lambda i: (0, 0)),
        pl.BlockSpec((1, F), lambda i: (0, 0)),
        pl.BlockSpec((F, F), lambda i: (0, 0)),
        pl.BlockSpec((1, F), lambda i: (0, 0)),
    ],
    out_specs=pl.BlockSpec((2, BE, D), lambda i: (0, i, 0)),
    out_shape=jax.ShapeDtypeStruct((2, E, D), jnp.float32),
)


def _post_body(a0_ref, a1_ref, x_ref, l2a_ref, l2b_ref, b2_ref, lw_ref, lb_ref,
               o_ref):
    t = (jnp.dot(a0_ref[...], l2a_ref[...], preferred_element_type=jnp.float32)
         + jnp.dot(a1_ref[...], l2b_ref[...], preferred_element_type=jnp.float32)
         + b2_ref[...])
    conv = jnp.dot(_ssp(t), lw_ref[...],
                   preferred_element_type=jnp.float32) + lb_ref[...]
    o_ref[...] = jnp.maximum(conv, 0.0) + x_ref[...]


_post_call = pl.pallas_call(
    _post_body,
    grid=(NP // BN,),
    in_specs=[
        pl.BlockSpec((BN, D), lambda i: (i, 0)),
        pl.BlockSpec((BN, D), lambda i: (i, 0)),
        pl.BlockSpec((BN, D), lambda i: (i, 0)),
        pl.BlockSpec((D, D), lambda i: (0, 0)),
        pl.BlockSpec((D, D), lambda i: (0, 0)),
        pl.BlockSpec((1, D), lambda i: (0, 0)),
        pl.BlockSpec((D, D), lambda i: (0, 0)),
        pl.BlockSpec((1, D), lambda i: (0, 0)),
    ],
    out_specs=pl.BlockSpec((BN, D), lambda i: (i, 0)),
    out_shape=jax.ShapeDtypeStruct((NP, D), jnp.float32),
)


def _edge_body(ea_ref, g_ref, wa_ref, wb_ref, b_ref, o_ref):
    t = (jnp.dot(ea_ref[...], wa_ref[...], preferred_element_type=jnp.float32)
         + jnp.dot(g_ref[...], wb_ref[...], preferred_element_type=jnp.float32)
         + b_ref[...])
    o_ref[...] = jnp.tanh(t) + ea_ref[...]


_edge_call = pl.pallas_call(
    _edge_body,
    grid=(E // BE,),
    in_specs=[
        pl.BlockSpec((BE, D), lambda i: (i, 0)),
        pl.BlockSpec((BE, D), lambda i: (i, 0)),
        pl.BlockSpec((D, D), lambda i: (0, 0)),
        pl.BlockSpec((D, D), lambda i: (0, 0)),
        pl.BlockSpec((1, D), lambda i: (0, 0)),
    ],
    out_specs=pl.BlockSpec((BE, D), lambda i: (i, 0)),
    out_shape=jax.ShapeDtypeStruct((E, D), jnp.float32),
)


@jax.jit
def kernel(x, edge_index, edge_attr, x_pos, mlp_w1, mlp_b1, mlp_w2, mlp_b2,
           lin1_w, lin2_w, lin2_b, lin_w, lin_b, emlp_w, emlp_b):
    row = edge_index[0]
    col = edge_index[1]
    row_sc = row.reshape(NS, _NSB, _SB, EB)
    col_sc = col.reshape(NS, _NSB, _SB, EB)
    row_eg = row.reshape(NC * NS, _NBLK_EG, EB)
    col_eg = col.reshape(NC * NS, _NBLK_EG, EB)
    pos4 = jnp.pad(x_pos, ((0, 0), (0, 1)))
    xp = jnp.pad(x, ((0, NP - N), (0, 0)))

    s = _sqdist(pos4.reshape(-1), row, col)
    cfac = _cut_call(s.reshape(E // 128, 128)).reshape(E, 1)
    h2 = _h_call(xp, lin1_w).reshape(2 * NP, D)
    w2 = _w_call(edge_attr, cfac, mlp_w1, mlp_b1.reshape(1, F),
                 mlp_w2, mlp_b2.reshape(1, F)).reshape(2 * E, D)
    agg2 = _scatter(h2, w2, row_sc, col_sc)
    xo = _post_call(agg2[:NP], agg2[NP:], xp, lin2_w[:D], lin2_w[D:],
                    lin2_b.reshape(1, D), lin_w, lin_b.reshape(1, D))
    g = _egather(xo, row_eg, col_eg)
    eo = _edge_call(edge_attr, g, emlp_w[:D], emlp_w[D:],
                    emlp_b.reshape(1, D))
    return (xo[:N], eo)
